# baseline (speedup 1.0000x reference)
import functools

import jax
import jax.numpy as jnp
from jax import lax
from jax.experimental import pallas as pl
from jax.experimental.pallas import tpu as pltpu
from jax.experimental.pallas import tpu_sc as plsc

EMB_DIM = 64
SCALE = 8.0

NUM_CORES = 2
NUM_SUBCORES = 16
NUM_WORKERS = NUM_CORES * NUM_SUBCORES

HALF = 64
NBUF = 8

NC = 8192
TB = 4


def _tc_transpose(table_t):
    d, v = table_t.shape

    def body(in_ref, out_ref):
        eye = jax.lax.broadcasted_iota(jnp.int32, (d, d), 0)
        eye = jnp.where(
            eye == jax.lax.broadcasted_iota(jnp.int32, (d, d), 1), 1.0, 0.0
        ).astype(jnp.float32)
        t = jax.lax.dot_general(
            in_ref[...], eye, (((0,), (0,)), ((), ())),
            preferred_element_type=jnp.float32,
            precision=jax.lax.Precision.HIGHEST)
        t3 = t.reshape(NC // 2, 2, d)
        out_ref[:, 0:d] = t3[:, 0, :]
        out_ref[:, d : 2 * d] = t3[:, 1, :]

    return pl.pallas_call(
        body,
        grid=(pl.cdiv(v, NC),),
        in_specs=[pl.BlockSpec((d, NC), lambda i: (0, i))],
        out_specs=pl.BlockSpec((NC // 2, 2 * d), lambda i: (i, 0)),
        out_shape=jax.ShapeDtypeStruct((v // 2, 2 * d), jnp.float32),
    )(table_t)


def _tc_pack(outg, s0, s1):
    hb = s0 // 2

    def body(in_ref, out_ref):
        for t in range(TB):
            sub = in_ref[t * hb : (t + 1) * hb, :]
            out_ref[t, :, 0:hb] = sub[:, 0:EMB_DIM].T * SCALE
            out_ref[t, :, hb : 2 * hb] = sub[:, EMB_DIM:128].T * SCALE

    return pl.pallas_call(
        body,
        grid=(s1 // TB,),
        in_specs=[pl.BlockSpec((TB * hb, 128), lambda i: (i, 0))],
        out_specs=pl.BlockSpec((TB, EMB_DIM, s0), lambda i: (i, 0, 0)),
        out_shape=jax.ShapeDtypeStruct((s1, EMB_DIM, s0), jnp.float32),
    )(outg)


def _make_gather(s0, s1):
    hb = s0 // 2
    assert hb == HALF * NUM_WORKERS and s1 % NBUF == 0
    mesh = plsc.VectorSubcoreMesh(core_axis_name="c", subcore_axis_name="s")

    @functools.partial(
        pl.kernel,
        mesh=mesh,
        out_type=jax.ShapeDtypeStruct((s0 * s1, EMB_DIM), jnp.float32),
        scratch_types=[
            pltpu.VMEM((s1, 2, HALF), jnp.int32),
            pltpu.VMEM((NBUF, 2 * HALF), jnp.int32),
            pltpu.VMEM((NBUF, 2 * HALF, EMB_DIM), jnp.float32),
            [pltpu.SemaphoreType.DMA] * NBUF,
            [pltpu.SemaphoreType.DMA] * NBUF,
        ],
        compiler_params=pltpu.CompilerParams(
            use_tc_tiling_on_sc=False, needs_layout_passes=False),
    )
    def gather_kernel(table_hbm, xt_hbm, out_hbm, ibuf, idx_v, g_v,
                      gsems, ssems):
        wid = lax.axis_index("s") * NUM_CORES + lax.axis_index("c")
        u0 = wid * HALF

        pltpu.sync_copy(xt_hbm.at[:, pl.ds(u0, HALF)], ibuf.at[:, 0])
        pltpu.sync_copy(xt_hbm.at[:, pl.ds(hb + u0, HALF)], ibuf.at[:, 1])

        lanes = lax.iota(jnp.int32, 16)
        h_vec = lanes & 1
        j_half = lax.shift_right_logical(lanes, 1)

        def start_gather(s, t):
            t_vec = jnp.full((16,), 0, jnp.int32) + t
            for k in range(2 * HALF // 16):
                idx_v.at[s][pl.ds(16 * k, 16)] = plsc.load_gather(
                    ibuf, [t_vec, h_vec, j_half + 8 * k])
            pltpu.make_async_copy(
                table_hbm.at[idx_v.at[s]], g_v.at[s], gsems[s]
            ).start()

        def process(s, t, prefetch):
            pltpu.make_async_copy(
                table_hbm.at[idx_v.at[s]], g_v.at[s], gsems[s]
            ).wait()
            dst = out_hbm.at[pl.ds(2 * (t * hb + u0), 2 * HALF)]
            pltpu.make_async_copy(g_v.at[s], dst, ssems[s]).start()
            pltpu.make_async_copy(g_v.at[s], dst, ssems[s]).wait()
            if prefetch:
                start_gather(s, t + NBUF)

        for s in range(NBUF):
            start_gather(s, s)

        def body(i, _):
            t = i * NBUF
            for s in range(NBUF):
                process(s, t + s, prefetch=True)
            return 0

        n_groups = s1 // NBUF
        lax.fori_loop(0, n_groups - 1, body, 0, unroll=False)
        tail = (n_groups - 1) * NBUF
        for s in range(NBUF):
            process(s, tail + s, prefetch=False)

    return gather_kernel


def kernel(x, table):
    S0, S1 = x.shape
    t2 = _tc_transpose(table.T)
    t4 = t2.reshape(table.shape[0], EMB_DIM)
    xt = x.T
    outg = _make_gather(S0, S1)(t4, xt)
    outp = _tc_pack(outg.reshape(S0 * S1 // 2, 128), S0, S1)
    return outp.transpose(2, 0, 1)


# speedup vs baseline: 1.4963x; 1.4963x over previous
import functools

import jax
import jax.numpy as jnp
from jax import lax
from jax.experimental import pallas as pl
from jax.experimental.pallas import tpu as pltpu
from jax.experimental.pallas import tpu_sc as plsc

EMB_DIM = 64
SCALE = 8.0

NUM_CORES = 2
NUM_SUBCORES = 16
NUM_WORKERS = NUM_CORES * NUM_SUBCORES

HALF = 64
NBUF = 8

NC = 16384
TB = 4


def _tc_transpose(table_t):
    d, v = table_t.shape

    def body(in_ref, out_ref):
        t = in_ref[...].T
        t3 = t.reshape(NC // 2, 2, d)
        out_ref[:, 0:d] = t3[:, 0, :]
        out_ref[:, d : 2 * d] = t3[:, 1, :]

    return pl.pallas_call(
        body,
        grid=(pl.cdiv(v, NC),),
        in_specs=[pl.BlockSpec((d, NC), lambda i: (0, i))],
        out_specs=pl.BlockSpec((NC // 2, 2 * d), lambda i: (i, 0)),
        out_shape=jax.ShapeDtypeStruct((v // 2, 2 * d), jnp.float32),
        compiler_params=pltpu.CompilerParams(vmem_limit_bytes=100 * 2**20),
    )(table_t)


def _tc_pack(outg, s0, s1):
    hb = s0 // 2

    def body(in_ref, out_ref):
        for t in range(TB):
            sub = in_ref[t * hb : (t + 1) * hb, :]
            out_ref[t, :, 0:hb] = sub[:, 0:EMB_DIM].T * SCALE
            out_ref[t, :, hb : 2 * hb] = sub[:, EMB_DIM:128].T * SCALE

    return pl.pallas_call(
        body,
        grid=(s1 // TB,),
        in_specs=[pl.BlockSpec((TB * hb, 128), lambda i: (i, 0))],
        out_specs=pl.BlockSpec((TB, EMB_DIM, s0), lambda i: (i, 0, 0)),
        out_shape=jax.ShapeDtypeStruct((s1, EMB_DIM, s0), jnp.float32),
    )(outg)


def _make_gather(s0, s1):
    hb = s0 // 2
    assert hb == HALF * NUM_WORKERS and s1 % NBUF == 0
    mesh = plsc.VectorSubcoreMesh(core_axis_name="c", subcore_axis_name="s")

    @functools.partial(
        pl.kernel,
        mesh=mesh,
        out_type=jax.ShapeDtypeStruct((s0 * s1, EMB_DIM), jnp.float32),
        scratch_types=[
            pltpu.VMEM((s1, 2, HALF), jnp.int32),
            pltpu.VMEM((NBUF, 2 * HALF), jnp.int32),
            pltpu.VMEM((NBUF, 2 * HALF, EMB_DIM), jnp.float32),
            [pltpu.SemaphoreType.DMA] * NBUF,
            [pltpu.SemaphoreType.DMA] * NBUF,
        ],
        compiler_params=pltpu.CompilerParams(
            use_tc_tiling_on_sc=False, needs_layout_passes=False),
    )
    def gather_kernel(table_hbm, xt_hbm, out_hbm, ibuf, idx_v, g_v,
                      gsems, ssems):
        wid = lax.axis_index("s") * NUM_CORES + lax.axis_index("c")
        u0 = wid * HALF

        pltpu.sync_copy(xt_hbm.at[:, pl.ds(u0, HALF)], ibuf.at[:, 0])
        pltpu.sync_copy(xt_hbm.at[:, pl.ds(hb + u0, HALF)], ibuf.at[:, 1])

        lanes = lax.iota(jnp.int32, 16)
        h_vec = lanes & 1
        j_half = lax.shift_right_logical(lanes, 1)

        def start_gather(s, t):
            t_vec = jnp.full((16,), 0, jnp.int32) + t
            for k in range(2 * HALF // 16):
                idx_v.at[s][pl.ds(16 * k, 16)] = plsc.load_gather(
                    ibuf, [t_vec, h_vec, j_half + 8 * k])
            pltpu.make_async_copy(
                table_hbm.at[idx_v.at[s]], g_v.at[s], gsems[s]
            ).start()

        def process(s, t, prefetch):
            pltpu.make_async_copy(
                table_hbm.at[idx_v.at[s]], g_v.at[s], gsems[s]
            ).wait()
            dst = out_hbm.at[pl.ds(2 * (t * hb + u0), 2 * HALF)]
            pltpu.make_async_copy(g_v.at[s], dst, ssems[s]).start()
            pltpu.make_async_copy(g_v.at[s], dst, ssems[s]).wait()
            if prefetch:
                start_gather(s, t + NBUF)

        for s in range(NBUF):
            start_gather(s, s)

        def body(i, _):
            t = i * NBUF
            for s in range(NBUF):
                process(s, t + s, prefetch=True)
            return 0

        n_groups = s1 // NBUF
        lax.fori_loop(0, n_groups - 1, body, 0, unroll=False)
        tail = (n_groups - 1) * NBUF
        for s in range(NBUF):
            process(s, tail + s, prefetch=False)

    return gather_kernel


def kernel(x, table):
    S0, S1 = x.shape
    t2 = _tc_transpose(table.T)
    t4 = t2.reshape(table.shape[0], EMB_DIM)
    xt = x.T
    outg = _make_gather(S0, S1)(t4, xt)
    outp = _tc_pack(outg.reshape(S0 * S1 // 2, 128), S0, S1)
    return outp.transpose(2, 0, 1)


# speedup vs baseline: 1.5183x; 1.0147x over previous
import functools

import jax
import jax.numpy as jnp
from jax import lax
from jax.experimental import pallas as pl
from jax.experimental.pallas import tpu as pltpu
from jax.experimental.pallas import tpu_sc as plsc

EMB_DIM = 64
SCALE = 8.0

NUM_CORES = 2
NUM_SUBCORES = 16
NUM_WORKERS = NUM_CORES * NUM_SUBCORES

HALF = 64
NBUF = 8

NC = 16384
TB = 8


def _tc_transpose(table_t):
    d, v = table_t.shape

    def body(in_ref, out_ref):
        t = in_ref[...].T
        t3 = t.reshape(NC // 2, 2, d)
        out_ref[:, 0:d] = t3[:, 0, :]
        out_ref[:, d : 2 * d] = t3[:, 1, :]

    return pl.pallas_call(
        body,
        grid=(pl.cdiv(v, NC),),
        in_specs=[pl.BlockSpec((d, NC), lambda i: (0, i))],
        out_specs=pl.BlockSpec((NC // 2, 2 * d), lambda i: (i, 0)),
        out_shape=jax.ShapeDtypeStruct((v // 2, 2 * d), jnp.float32),
        compiler_params=pltpu.CompilerParams(vmem_limit_bytes=100 * 2**20),
    )(table_t)


def _tc_pack(outg, s0, s1):
    hb = s0 // 2

    def body(in_ref, out_ref):
        for t in range(TB):
            sub = in_ref[t * hb : (t + 1) * hb, :]
            out_ref[t, :, 0:hb] = sub[:, 0:EMB_DIM].T * SCALE
            out_ref[t, :, hb : 2 * hb] = sub[:, EMB_DIM:128].T * SCALE

    return pl.pallas_call(
        body,
        grid=(s1 // TB,),
        in_specs=[pl.BlockSpec((TB * hb, 128), lambda i: (i, 0))],
        out_specs=pl.BlockSpec((TB, EMB_DIM, s0), lambda i: (i, 0, 0)),
        out_shape=jax.ShapeDtypeStruct((s1, EMB_DIM, s0), jnp.float32),
        compiler_params=pltpu.CompilerParams(vmem_limit_bytes=100 * 2**20),
    )(outg)


def _make_gather(s0, s1):
    hb = s0 // 2
    assert hb == HALF * NUM_WORKERS and s1 % NBUF == 0
    mesh = plsc.VectorSubcoreMesh(core_axis_name="c", subcore_axis_name="s")

    @functools.partial(
        pl.kernel,
        mesh=mesh,
        out_type=jax.ShapeDtypeStruct((s0 * s1, EMB_DIM), jnp.float32),
        scratch_types=[
            pltpu.VMEM((s1, 2, HALF), jnp.int32),
            pltpu.VMEM((NBUF, 2 * HALF), jnp.int32),
            pltpu.VMEM((NBUF, 2 * HALF, EMB_DIM), jnp.float32),
            [pltpu.SemaphoreType.DMA] * NBUF,
            [pltpu.SemaphoreType.DMA] * NBUF,
        ],
        compiler_params=pltpu.CompilerParams(
            use_tc_tiling_on_sc=False, needs_layout_passes=False),
    )
    def gather_kernel(table_hbm, xt_hbm, out_hbm, ibuf, idx_v, g_v,
                      gsems, ssems):
        wid = lax.axis_index("s") * NUM_CORES + lax.axis_index("c")
        u0 = wid * HALF

        pltpu.sync_copy(xt_hbm.at[:, pl.ds(u0, HALF)], ibuf.at[:, 0])
        pltpu.sync_copy(xt_hbm.at[:, pl.ds(hb + u0, HALF)], ibuf.at[:, 1])

        lanes = lax.iota(jnp.int32, 16)
        h_vec = lanes & 1
        j_half = lax.shift_right_logical(lanes, 1)

        def start_gather(s, t):
            t_vec = jnp.full((16,), 0, jnp.int32) + t
            for k in range(2 * HALF // 16):
                idx_v.at[s][pl.ds(16 * k, 16)] = plsc.load_gather(
                    ibuf, [t_vec, h_vec, j_half + 8 * k])
            pltpu.make_async_copy(
                table_hbm.at[idx_v.at[s]], g_v.at[s], gsems[s]
            ).start()

        def process(s, t, prefetch):
            pltpu.make_async_copy(
                table_hbm.at[idx_v.at[s]], g_v.at[s], gsems[s]
            ).wait()
            dst = out_hbm.at[pl.ds(2 * (t * hb + u0), 2 * HALF)]
            pltpu.make_async_copy(g_v.at[s], dst, ssems[s]).start()
            pltpu.make_async_copy(g_v.at[s], dst, ssems[s]).wait()
            if prefetch:
                start_gather(s, t + NBUF)

        for s in range(NBUF):
            start_gather(s, s)

        def body(i, _):
            t = i * NBUF
            for s in range(NBUF):
                process(s, t + s, prefetch=True)
            return 0

        n_groups = s1 // NBUF
        lax.fori_loop(0, n_groups - 1, body, 0, unroll=False)
        tail = (n_groups - 1) * NBUF
        for s in range(NBUF):
            process(s, tail + s, prefetch=False)

    return gather_kernel


def kernel(x, table):
    S0, S1 = x.shape
    t2 = _tc_transpose(table.T)
    t4 = t2.reshape(table.shape[0], EMB_DIM)
    xt = x.T
    outg = _make_gather(S0, S1)(t4, xt)
    outp = _tc_pack(outg.reshape(S0 * S1 // 2, 128), S0, S1)
    return outp.transpose(2, 0, 1)
